# Initial kernel scaffold; baseline (speedup 1.0000x reference)
import jax
import jax.numpy as jnp
from jax.experimental import pallas as pl


def kernel(feats, node_neigh, W_prep, W0, W1):
    raise NotImplementedError("write your pallas kernel here")


# speedup vs baseline: 1.7855x; 1.7855x over previous
import functools

import jax
import jax.numpy as jnp
from jax import lax
from jax.experimental import pallas as pl
from jax.experimental.pallas import tpu as pltpu
from jax.experimental.pallas import tpu_sc as plsc

_NC = 2
_NS = 16
_NW = _NC * _NS
_C = 64


def _gather_sum(table, neigh_c, npad):
    nq, s, c = neigh_c.shape
    d = table.shape[1]
    npw = npad // _NW
    nch = npw // _C
    mesh = plsc.VectorSubcoreMesh(core_axis_name="c", subcore_axis_name="s")

    @functools.partial(
        pl.kernel,
        out_type=jax.ShapeDtypeStruct((npad, d), jnp.float32),
        mesh=mesh,
        scratch_types=[
            pltpu.VMEM((s, _C), jnp.int32),
            pltpu.VMEM((_C, d), jnp.float32),
            pltpu.SemaphoreType.DMA,
            pltpu.SemaphoreType.DMA,
        ],
    )
    def gather_kernel(table_hbm, neigh_hbm, out_hbm, idx_v, acc_v, sem0, sem):
        wid = lax.axis_index("s") * _NC + lax.axis_index("c")

        def chunk(ch, carry):
            q = wid * nch + ch
            base = q * _C
            pltpu.sync_copy(neigh_hbm.at[q], idx_v)
            pltpu.async_copy(table_hbm.at[idx_v.at[0]], acc_v, sem0).wait()
            cps = [
                pltpu.async_copy(table_hbm.at[idx_v.at[j]], acc_v, sem, add=True)
                for j in range(1, s)
            ]
            for cp in cps:
                cp.wait()
            pltpu.sync_copy(acc_v, out_hbm.at[pl.ds(base, _C)])
            return carry

        lax.fori_loop(0, nch, chunk, 0)

    return gather_kernel(table, neigh_c)


def _matmul(x, w):
    def body(x_ref, w_ref, o_ref):
        o_ref[...] = jnp.dot(x_ref[...], w_ref[...],
                             preferred_element_type=jnp.float32)

    return pl.pallas_call(
        body,
        out_shape=jax.ShapeDtypeStruct((x.shape[0], w.shape[1]), jnp.float32),
    )(x, w)


def _layer0(x, agg_sum, w_self, w_neigh, scale):
    def body(x_ref, s_ref, wa_ref, wb_ref, o_ref):
        m = jnp.dot(x_ref[...], wa_ref[...], preferred_element_type=jnp.float32)
        m = m + jnp.dot(s_ref[...] * scale, wb_ref[...],
                        preferred_element_type=jnp.float32)
        o_ref[...] = jnp.maximum(m, 0.0)

    return pl.pallas_call(
        body,
        out_shape=jax.ShapeDtypeStruct((x.shape[0], w_self.shape[1]), jnp.float32),
    )(x, agg_sum, w_self, w_neigh)


def _layer1(h0, agg_sum, w_self, w_neigh, scale):
    h = h0.shape[1]

    def body(h_ref, s_ref, wa_ref, wb_ref, o_ref):
        m = jnp.dot(h_ref[...], wa_ref[...], preferred_element_type=jnp.float32)
        m = m + jnp.dot(s_ref[...] * scale, wb_ref[...],
                        preferred_element_type=jnp.float32)
        o_ref[:, :h] = h_ref[...]
        o_ref[:, h:] = jnp.maximum(m, 0.0)

    return pl.pallas_call(
        body,
        out_shape=jax.ShapeDtypeStruct(
            (h0.shape[0], h + w_self.shape[1]), jnp.float32),
    )(h0, agg_sum, w_self, w_neigh)


def kernel(feats, node_neigh, W_prep, W0, W1):
    n, s = node_neigh.shape
    p = W_prep.shape[1]
    h0_dim = W0.shape[1]
    scale = 1.0 / s

    npad = -(-n // (_NW * _C)) * (_NW * _C)
    neigh_t = jnp.pad(node_neigh.T, ((0, 0), (0, npad - n)))
    neigh_c = neigh_t.reshape(s, npad // _C, _C).transpose(1, 0, 2)

    all_feats = _matmul(feats, W_prep)
    s0 = _gather_sum(all_feats, neigh_c, npad)[:n]
    h0 = _layer0(all_feats, s0, W0[:p], W0[p:], scale)
    s1 = _gather_sum(h0, neigh_c, npad)[:n]
    out = _layer1(h0, s1, W1[:h0_dim], W1[h0_dim:], scale)
    return out[None]


# speedup vs baseline: 1.8307x; 1.0253x over previous
import functools

import jax
import jax.numpy as jnp
from jax import lax
from jax.experimental import pallas as pl
from jax.experimental.pallas import tpu as pltpu
from jax.experimental.pallas import tpu_sc as plsc

_NC = 2
_NS = 16
_NW = _NC * _NS
_C = 64


def _gather_sum(table, neigh_c, npad):
    nq, s, c = neigh_c.shape
    d = table.shape[1]
    k0 = (nq * 13) // 16
    mesh = plsc.VectorSubcoreMesh(core_axis_name="c", subcore_axis_name="s")

    @functools.partial(
        pl.kernel,
        out_type=jax.ShapeDtypeStruct((npad, d), jnp.float32),
        mesh=mesh,
        scratch_types=[
            pltpu.VMEM((s, _C), jnp.int32),
            pltpu.VMEM((_C, d), jnp.float32),
            pltpu.SemaphoreType.DMA,
            pltpu.SemaphoreType.DMA,
        ],
    )
    def gather_kernel(table_hbm, neigh_hbm, out_hbm, idx_v, acc_v, sem0, sem):
        cid = lax.axis_index("c")
        sid = lax.axis_index("s")
        k1 = nq - k0
        lo = jnp.where(cid == 0, (sid * k0) // _NS, k0 + (sid * k1) // _NS)
        hi = jnp.where(cid == 0, ((sid + 1) * k0) // _NS,
                       k0 + ((sid + 1) * k1) // _NS)

        def chunk(q, carry):
            base = q * _C
            pltpu.sync_copy(neigh_hbm.at[q], idx_v)
            pltpu.async_copy(table_hbm.at[idx_v.at[0]], acc_v, sem0).wait()
            cps = [
                pltpu.async_copy(table_hbm.at[idx_v.at[j]], acc_v, sem, add=True)
                for j in range(1, s)
            ]
            for cp in cps:
                cp.wait()
            pltpu.sync_copy(acc_v, out_hbm.at[pl.ds(base, _C)])
            return carry

        lax.fori_loop(lo, hi, chunk, 0)

    return gather_kernel(table, neigh_c)


def _matmul(x, w):
    def body(x_ref, w_ref, o_ref):
        o_ref[...] = jnp.dot(x_ref[...], w_ref[...],
                             preferred_element_type=jnp.float32)

    return pl.pallas_call(
        body,
        out_shape=jax.ShapeDtypeStruct((x.shape[0], w.shape[1]), jnp.float32),
    )(x, w)


def _layer0(x, agg_sum, w_self, w_neigh, scale):
    def body(x_ref, s_ref, wa_ref, wb_ref, o_ref):
        m = jnp.dot(x_ref[...], wa_ref[...], preferred_element_type=jnp.float32)
        m = m + jnp.dot(s_ref[...] * scale, wb_ref[...],
                        preferred_element_type=jnp.float32)
        o_ref[...] = jnp.maximum(m, 0.0)

    return pl.pallas_call(
        body,
        out_shape=jax.ShapeDtypeStruct((x.shape[0], w_self.shape[1]), jnp.float32),
    )(x, agg_sum, w_self, w_neigh)


def _layer1(h0, agg_sum, w_self, w_neigh, scale):
    h = h0.shape[1]

    def body(h_ref, s_ref, wa_ref, wb_ref, o_ref):
        m = jnp.dot(h_ref[...], wa_ref[...], preferred_element_type=jnp.float32)
        m = m + jnp.dot(s_ref[...] * scale, wb_ref[...],
                        preferred_element_type=jnp.float32)
        o_ref[:, :h] = h_ref[...]
        o_ref[:, h:] = jnp.maximum(m, 0.0)

    return pl.pallas_call(
        body,
        out_shape=jax.ShapeDtypeStruct(
            (h0.shape[0], h + w_self.shape[1]), jnp.float32),
    )(h0, agg_sum, w_self, w_neigh)


def kernel(feats, node_neigh, W_prep, W0, W1):
    n, s = node_neigh.shape
    p = W_prep.shape[1]
    h0_dim = W0.shape[1]
    scale = 1.0 / s

    npad = -(-n // (_NW * _C)) * (_NW * _C)
    neigh_t = jnp.pad(node_neigh.T, ((0, 0), (0, npad - n)))
    neigh_c = neigh_t.reshape(s, npad // _C, _C).transpose(1, 0, 2)

    all_feats = _matmul(feats, W_prep)
    s0 = _gather_sum(all_feats, neigh_c, npad)[:n]
    h0 = _layer0(all_feats, s0, W0[:p], W0[p:], scale)
    s1 = _gather_sum(h0, neigh_c, npad)[:n]
    out = _layer1(h0, s1, W1[:h0_dim], W1[h0_dim:], scale)
    return out[None]
